# baseline (speedup 1.0000x reference)
import functools

import jax
import jax.numpy as jnp
from jax import lax
from jax.experimental import pallas as pl
from jax.experimental.pallas import tpu as pltpu
from jax.experimental.pallas import tpu_sc as plsc

D = 128
H = 8
DH = D // H
N_HIGH = 512
E_HIGH = 8192
N_LOW = 10000
E_LOW = 320000

NC = 2
NS = 16
NW = NC * NS
C = 64
CH = C // 2
NCHUNK = E_LOW // C
TROWS = 10112
DROWS = 632
HI = jax.lax.Precision.HIGHEST


def _pre_body(x_ref, w_ref, b_ref, wq_ref, bq_ref, wk_ref, bk_ref,
              wv_ref, bv_ref, ws_ref, bs_ref,
              tq_ref, tkv_ref, skip_ref):
    x = x_ref[...]
    mu = jnp.mean(x, axis=-1, keepdims=True)
    var = jnp.mean((x - mu) ** 2, axis=-1, keepdims=True)
    ln = (x - mu) / jnp.sqrt(var + 1e-5) * w_ref[...] + b_ref[...]
    dot = lambda a, w: lax.dot_general(a, w, (((1,), (1,)), ((), ())),
                                       precision=HI)
    tq_ref[...] = (dot(ln, wq_ref[...]) + bq_ref[...]) * 0.25
    tkv_ref[:, :D] = dot(ln, wk_ref[...]) + bk_ref[...]
    tkv_ref[:, D:] = dot(ln, wv_ref[...]) + bv_ref[...]
    skip_ref[...] = dot(ln, ws_ref[...]) + bs_ref[...]


def _pre(low_emb_in, norm_w, norm_b, Wq, bq, Wk, bk, Wv, bv, Ws, bs):
    rows = 1000
    grid = N_LOW // rows
    full = pl.BlockSpec((D, D), lambda i: (0, 0))
    vec = pl.BlockSpec((1, D), lambda i: (0, 0))
    chunk = pl.BlockSpec((rows, D), lambda i: (i, 0))
    kchunk = pl.BlockSpec((rows, 2 * D), lambda i: (i, 0))
    return pl.pallas_call(
        _pre_body,
        grid=(grid,),
        in_specs=[chunk, vec, vec, full, vec, full, vec, full, vec, full, vec],
        out_specs=[chunk, kchunk, chunk],
        out_shape=[jax.ShapeDtypeStruct((N_LOW, D), jnp.float32),
                   jax.ShapeDtypeStruct((N_LOW, 2 * D), jnp.float32),
                   jax.ShapeDtypeStruct((N_LOW, D), jnp.float32)],
    )(low_emb_in, norm_w.reshape(1, D), norm_b.reshape(1, D),
      Wq, bq.reshape(1, D), Wk, bk.reshape(1, D), Wv, bv.reshape(1, D),
      Ws, bs.reshape(1, D))


def _highA_body(hs_ref, hd_ref, a_ref, acc):
    i = pl.program_id(0)

    @pl.when(i == 0)
    def _():
        acc[...] = jnp.zeros_like(acc)

    hs = hs_ref[0, 0, :]
    hd = hd_ref[0, 0, :]
    cols = lax.broadcasted_iota(jnp.int32, (1024, N_HIGH), 1)
    ms = (hs[:, None] == cols).astype(jnp.float32)
    md = (hd[:, None] == cols).astype(jnp.float32)
    acc[...] += lax.dot_general(md, ms, (((0,), (0,)), ((), ())),
                                precision=HI)

    @pl.when(i == pl.num_programs(0) - 1)
    def _():
        a_ref[...] = acc[...]


def _highA(high_edge_index):
    hs = high_edge_index[0].reshape(E_HIGH // 1024, 1, 1024)
    hd = high_edge_index[1].reshape(E_HIGH // 1024, 1, 1024)
    espec = pl.BlockSpec((1, 1, 1024), lambda i: (i, 0, 0))
    return pl.pallas_call(
        _highA_body,
        grid=(E_HIGH // 1024,),
        in_specs=[espec, espec],
        out_specs=pl.BlockSpec((N_HIGH, N_HIGH), lambda i: (0, 0)),
        out_shape=jax.ShapeDtypeStruct((N_HIGH, N_HIGH), jnp.float32),
        scratch_shapes=[pltpu.VMEM((N_HIGH, N_HIGH), jnp.float32)],
    )(hs, hd)


def _high2_body(x_ref, a_ref, w_ref, b_ref, eps_ref, ginw_ref, ginb_ref,
                wq_ref, bq_ref, wk_ref, bk_ref, wv_ref, bv_ref,
                wo_ref, bo_ref, out_ref):
    x = x_ref[...]
    mu = jnp.mean(x, axis=-1, keepdims=True)
    var = jnp.mean((x - mu) ** 2, axis=-1, keepdims=True)
    ln = (x - mu) / jnp.sqrt(var + 1e-5) * w_ref[...] + b_ref[...]
    dot = lambda a, w: lax.dot_general(a, w, (((1,), (1,)), ((), ())),
                                       precision=HI)
    agg = lax.dot_general(a_ref[...], ln, (((1,), (0,)), ((), ())),
                          precision=HI)
    gin = dot((1.0 + eps_ref[0, 0]) * ln + agg, ginw_ref[...]) + ginb_ref[...]
    q = dot(ln, wq_ref[...]) + bq_ref[...]
    k = dot(ln, wk_ref[...]) + bk_ref[...]
    v = dot(ln, wv_ref[...]) + bv_ref[...]
    outs = []
    for h in range(H):
        qh = q[:, h * DH:(h + 1) * DH]
        kh = k[:, h * DH:(h + 1) * DH]
        vh = v[:, h * DH:(h + 1) * DH]
        s = lax.dot_general(qh, kh, (((1,), (1,)), ((), ())),
                            precision=HI) * (DH ** -0.5)
        s = s - jnp.max(s, axis=-1, keepdims=True)
        e = jnp.exp(s)
        att = e / jnp.sum(e, axis=-1, keepdims=True)
        outs.append(lax.dot_general(att, vh, (((1,), (0,)), ((), ())),
                                    precision=HI))
    mh = jnp.concatenate(outs, axis=1)
    out_ref[...] = dot(mh, wo_ref[...]) + bo_ref[...] + gin


def _high2(high_emb_in, A, norm_w, norm_b, gin_eps, gin_W, gin_b,
           Wq, Wk, Wv, bq, bk, bv, Wo, bo):
    return pl.pallas_call(
        _high2_body,
        out_shape=jax.ShapeDtypeStruct((N_HIGH, D), jnp.float32),
    )(high_emb_in, A, norm_w.reshape(1, D), norm_b.reshape(1, D),
      gin_eps.reshape(1, 1), gin_W, gin_b.reshape(1, D),
      Wq, bq.reshape(1, D), Wk, bk.reshape(1, D), Wv, bv.reshape(1, D),
      Wo, bo.reshape(1, D))


def _sc_edge_body(tq_hbm, tkv_hbm, ls_hbm, ld_hbm,
                  num_hbm, den_hbm,
                  idx_s, idx_d, idx_dc, idx_den, qrows, kvrows, onum, dbuf,
                  ntab, dtab, sga, sgb, ssn, ssd):
    cid = lax.axis_index("c")
    sid = lax.axis_index("s")
    wid = sid * NC + cid

    lane = lax.iota(jnp.int32, 16)
    low8 = lane < 8
    zero16 = jnp.zeros((16,), jnp.float32)
    zero16i = jnp.zeros((16,), jnp.int32)

    def zrow(r, carry):
        rv = jnp.full((16,), r, jnp.int32)
        for cc in range(D // 16):
            plsc.store_scatter(dbuf, [rv, cc * 16 + lane], zero16)
            plsc.store_scatter(onum, [rv, cc * 16 + lane], zero16)
        return carry

    lax.fori_loop(0, C, zrow, 0)
    for g in range(C // 16):
        eidx = g * 16 + lane
        plsc.store_scatter(idx_dc, [eidx], zero16i)
        plsc.store_scatter(idx_den, [eidx], zero16i)
    tbase = sid * (TROWS // NS)
    for kk in range(TROWS // NS // C):
        pltpu.sync_copy(dbuf, ntab.at[pl.ds(tbase + kk * C, C)])
    pltpu.sync_copy(dbuf.at[pl.ds(0, TROWS // NS - (TROWS // NS // C) * C)],
                    ntab.at[pl.ds(tbase + (TROWS // NS // C) * C,
                                  TROWS // NS - (TROWS // NS // C) * C)])

    @pl.when(sid == 0)
    def _():
        for kk in range(DROWS // C):
            pltpu.sync_copy(dbuf, dtab.at[pl.ds(kk * C, C)])
        pltpu.sync_copy(dbuf.at[pl.ds(0, DROWS - (DROWS // C) * C)],
                        dtab.at[pl.ds((DROWS // C) * C,
                                      DROWS - (DROWS // C) * C)])

    plsc.subcore_barrier()

    pltpu.async_copy(onum, ntab.at[idx_dc], ssn, add=True)
    pltpu.async_copy(dbuf, dtab.at[idx_den], ssd, add=True)
    base0 = wid * C
    pltpu.sync_copy(ls_hbm.at[pl.ds(base0, C)], idx_s)
    pltpu.sync_copy(ld_hbm.at[pl.ds(base0, C)], idx_d)
    pltpu.async_copy(tq_hbm.at[idx_d.at[pl.ds(0, CH)]],
                     qrows.at[pl.ds(0, CH)], sga)
    pltpu.async_copy(tkv_hbm.at[idx_s.at[pl.ds(0, CH)]],
                     kvrows.at[pl.ds(0, CH)], sga)
    pltpu.async_copy(tq_hbm.at[idx_d.at[pl.ds(CH, CH)]],
                     qrows.at[pl.ds(CH, CH)], sgb)
    pltpu.async_copy(tkv_hbm.at[idx_s.at[pl.ds(CH, CH)]],
                     kvrows.at[pl.ds(CH, CH)], sgb)

    def wait_scatters():
        pltpu.make_async_copy(onum, ntab.at[idx_dc], ssn).wait()
        pltpu.make_async_copy(dbuf, dtab.at[idx_den], ssd).wait()

    def wait_gathers(sem, lo):
        pltpu.make_async_copy(tq_hbm.at[idx_d.at[pl.ds(lo, CH)]],
                              qrows.at[pl.ds(lo, CH)], sem).wait()
        pltpu.make_async_copy(tkv_hbm.at[idx_s.at[pl.ds(lo, CH)]],
                              kvrows.at[pl.ds(lo, CH)], sem).wait()

    def chunk(j, carry):
        wait_scatters()

        def groupC(g, gcarry):
            eidx = g * 16 + lane
            ldv = plsc.load_gather(idx_dc, [eidx])
            colb = lax.shift_left(ldv & 15, 3)
            for h in range(H):
                plsc.store_scatter(dbuf, [eidx, colb + h], zero16)
            return gcarry

        lax.fori_loop(0, C // 16, groupC, 0)

        def groupA(g, gcarry):
            base16 = g * 16
            eidx = base16 + lane
            ldv = plsc.load_gather(idx_d, [eidx])
            plsc.store_scatter(idx_dc, [eidx], ldv)
            plsc.store_scatter(idx_den, [eidx], lax.shift_right_logical(ldv, 4))
            colbv = lax.shift_left(ldv & 15, 3)
            for e in range(16):
                i = base16 + e
                sv = zero16
                for h in range(H):
                    sl = pl.ds(h * DH, DH)
                    p = qrows[i, sl] * kvrows[i, sl]
                    sv = jnp.where(lane == h, jnp.sum(p), sv)
                evm = jnp.where(low8, jnp.exp(sv), 0.0)
                plsc.store_scatter(dbuf, [jnp.full((16,), i, jnp.int32),
                                          colbv[e] + lane], evm, mask=low8)
                for h in range(H):
                    onum[i, pl.ds(h * DH, DH)] = (
                        kvrows[i, pl.ds(D + h * DH, DH)] * evm[h])
            return gcarry

        wait_gathers(sga, 0)
        lax.fori_loop(0, CH // 16, groupA, 0)
        wait_gathers(sgb, CH)
        lax.fori_loop(CH // 16, C // 16, groupA, 0)

        pltpu.async_copy(onum, ntab.at[idx_dc], ssn, add=True)
        pltpu.async_copy(dbuf, dtab.at[idx_den], ssd, add=True)

        nbase = jnp.minimum(((j + 1) * NW + wid) * C, E_LOW - C)
        pltpu.sync_copy(ls_hbm.at[pl.ds(nbase, C)], idx_s)
        pltpu.sync_copy(ld_hbm.at[pl.ds(nbase, C)], idx_d)
        pltpu.async_copy(tq_hbm.at[idx_d.at[pl.ds(0, CH)]],
                         qrows.at[pl.ds(0, CH)], sga)
        pltpu.async_copy(tkv_hbm.at[idx_s.at[pl.ds(0, CH)]],
                         kvrows.at[pl.ds(0, CH)], sga)
        pltpu.async_copy(tq_hbm.at[idx_d.at[pl.ds(CH, CH)]],
                         qrows.at[pl.ds(CH, CH)], sgb)
        pltpu.async_copy(tkv_hbm.at[idx_s.at[pl.ds(CH, CH)]],
                         kvrows.at[pl.ds(CH, CH)], sgb)
        return carry

    base_chunks = NCHUNK // NW
    extra = NCHUNK - base_chunks * NW
    nch = jnp.where(wid < extra, base_chunks + 1, base_chunks)
    lax.fori_loop(0, nch, chunk, 0)

    wait_gathers(sga, 0)
    wait_gathers(sgb, CH)
    wait_scatters()
    plsc.subcore_barrier()

    obase = sid * (TROWS // NS)
    pltpu.sync_copy(ntab.at[pl.ds(obase, TROWS // NS)],
                    num_hbm.at[cid, pl.ds(obase, TROWS // NS)])

    @pl.when(sid == 0)
    def _():
        pltpu.sync_copy(dtab, den_hbm.at[cid])


_sc_mesh = plsc.VectorSubcoreMesh(core_axis_name="c", subcore_axis_name="s")

_sc_edges = pl.kernel(
    _sc_edge_body,
    out_type=[jax.ShapeDtypeStruct((NC, TROWS, D), jnp.float32),
              jax.ShapeDtypeStruct((NC, DROWS, D), jnp.float32)],
    mesh=_sc_mesh,
    compiler_params=pltpu.CompilerParams(needs_layout_passes=False),
    scratch_types=[
        pltpu.VMEM((C,), jnp.int32),
        pltpu.VMEM((C,), jnp.int32),
        pltpu.VMEM((C,), jnp.int32),
        pltpu.VMEM((C,), jnp.int32),
        pltpu.VMEM((C, D), jnp.float32),
        pltpu.VMEM((C, 2 * D), jnp.float32),
        pltpu.VMEM((C, D), jnp.float32),
        pltpu.VMEM((C, D), jnp.float32),
        pltpu.VMEM_SHARED((TROWS, D), jnp.float32),
        pltpu.VMEM_SHARED((DROWS, D), jnp.float32),
        pltpu.SemaphoreType.DMA,
        pltpu.SemaphoreType.DMA,
        pltpu.SemaphoreType.DMA,
        pltpu.SemaphoreType.DMA,
    ],
)


def _gelu(x):
    return 0.5 * x * (1.0 + lax.erf(x * (2.0 ** -0.5)))


def _post_body(n0_ref, n1_ref, d0_ref, d1_ref, skip_ref, lb_ref, he2_ref,
               chlq_ref, chlk_ref, chlv_ref, clhq_ref, clhk_ref, clhv_ref,
               oh_ref, ol_ref, pooled, counts):
    i = pl.program_id(0)

    @pl.when(i == 0)
    def _():
        pooled[...] = jnp.zeros_like(pooled)
        counts[...] = jnp.zeros_like(counts)

    dot = lambda a, w: lax.dot_general(a, w, (((1,), (1,)), ((), ())),
                                       precision=HI)

    num = n0_ref[...] + n1_ref[...]
    den8 = d0_ref[...] + d1_ref[...]
    hrow = lax.broadcasted_iota(jnp.int32, (8, D), 0)
    hcol = lax.broadcasted_iota(jnp.int32, (8, D), 1) // DH
    expand = (hrow == hcol).astype(jnp.float32)
    den = lax.dot_general(den8, expand, (((1,), (0,)), ((), ())),
                          precision=HI)
    tout = num / jnp.maximum(den, 1e-30)
    low2 = tout + skip_ref[...]

    lb = lb_ref[0, 0, :]
    cols = lax.broadcasted_iota(jnp.int32, (lb.shape[0], N_HIGH), 1)
    P = (lb[:, None] == cols).astype(jnp.float32)
    pooled[...] += lax.dot_general(P, low2, (((0,), (0,)), ((), ())),
                                   precision=HI)
    counts[...] += lax.dot_general(P, jnp.ones_like(low2),
                                   (((0,), (0,)), ((), ())), precision=HI)

    he2 = he2_ref[...]
    hpn = lax.dot_general(P, he2, (((1,), (0,)), ((), ())), precision=HI)
    Q = dot(low2, clhq_ref[...])
    K = dot(hpn, clhk_ref[...])
    V = dot(low2, clhv_ref[...])
    w = jnp.sum(Q * K, axis=1, keepdims=True) * (float(D) ** -0.5)
    ol_ref[...] = _gelu(w * V)

    @pl.when(i == pl.num_programs(0) - 1)
    def _():
        x = pooled[...] / jnp.maximum(counts[...], 1.0)
        Qh = dot(he2, chlq_ref[...])
        Kh = dot(x, chlk_ref[...])
        Vh = dot(he2, chlv_ref[...])
        wh = jnp.sum(Qh * Kh, axis=1, keepdims=True) * (float(D) ** -0.5)
        oh_ref[...] = _gelu(wh * Vh)


def _post(num0, num1, den0, den1, skip, low_batch, high_emb2,
          chl_Q, chl_K, chl_V, clh_Q, clh_K, clh_V):
    rows = 2000
    grid = N_LOW // rows
    chunk = pl.BlockSpec((rows, D), lambda i: (i, 0))
    dchunk = pl.BlockSpec((rows, 8), lambda i: (i, 0))
    full = pl.BlockSpec((D, D), lambda i: (0, 0))
    hfull = pl.BlockSpec((N_HIGH, D), lambda i: (0, 0))
    lspec = pl.BlockSpec((1, 1, rows), lambda i: (i, 0, 0))
    lb3 = low_batch.reshape(grid, 1, rows)
    return pl.pallas_call(
        _post_body,
        grid=(grid,),
        in_specs=[chunk, chunk, dchunk, dchunk, chunk, lspec, hfull,
                  full, full, full, full, full, full],
        out_specs=[hfull, chunk],
        out_shape=[jax.ShapeDtypeStruct((N_HIGH, D), jnp.float32),
                   jax.ShapeDtypeStruct((N_LOW, D), jnp.float32)],
        scratch_shapes=[pltpu.VMEM((N_HIGH, D), jnp.float32),
                        pltpu.VMEM((N_HIGH, D), jnp.float32)],
    )(num0, num1, den0, den1, skip, lb3, high_emb2,
      chl_Q, chl_K, chl_V, clh_Q, clh_K, clh_V)


def kernel(high_emb_in, high_edge_index, low_emb_in, low_edge_index,
           low_batch, norm_w, norm_b, gin_eps, gin_W, gin_b,
           mha_Wq, mha_Wk, mha_Wv, mha_bq, mha_bk, mha_bv, mha_Wo, mha_bo,
           tc_Wq, tc_bq, tc_Wk, tc_bk, tc_Wv, tc_bv, tc_Wskip, tc_bskip,
           chl_Q, chl_K, chl_V, clh_Q, clh_K, clh_V):
    tq, tkv, skip = _pre(low_emb_in, norm_w, norm_b,
                         tc_Wq, tc_bq, tc_Wk, tc_bk, tc_Wv, tc_bv,
                         tc_Wskip, tc_bskip)
    A = _highA(high_edge_index)
    high_emb2 = _high2(high_emb_in, A, norm_w, norm_b, gin_eps, gin_W, gin_b,
                       mha_Wq, mha_Wk, mha_Wv, mha_bq, mha_bk, mha_bv,
                       mha_Wo, mha_bo)
    ls = low_edge_index[0]
    ld = low_edge_index[1]
    num_p, den_p = _sc_edges(tq, tkv, ls, ld)
    den_lin = den_p.reshape(NC, DROWS * 16, 8)
    out_high, out_low = _post(num_p[0, :N_LOW], num_p[1, :N_LOW],
                              den_lin[0, :N_LOW], den_lin[1, :N_LOW],
                              skip, low_batch, high_emb2,
                              chl_Q, chl_K, chl_V, clh_Q, clh_K, clh_V)
    return (out_high, out_low)


# speedup vs baseline: 1.0825x; 1.0825x over previous
import functools

import jax
import jax.numpy as jnp
from jax import lax
from jax.experimental import pallas as pl
from jax.experimental.pallas import tpu as pltpu
from jax.experimental.pallas import tpu_sc as plsc

D = 128
H = 8
DH = D // H
N_HIGH = 512
E_HIGH = 8192
N_LOW = 10000
E_LOW = 320000

NC = 2
NS = 16
NW = NC * NS
C = 64
CH = C // 2
NCHUNK = E_LOW // C
TROWS = 10112
DROWS = 632
HI = jax.lax.Precision.HIGHEST


def _pre_body(x_ref, w_ref, b_ref, wq_ref, bq_ref, wk_ref, bk_ref,
              wv_ref, bv_ref, ws_ref, bs_ref,
              tq_ref, tkv_ref, skip_ref):
    x = x_ref[...]
    mu = jnp.mean(x, axis=-1, keepdims=True)
    var = jnp.mean((x - mu) ** 2, axis=-1, keepdims=True)
    ln = (x - mu) / jnp.sqrt(var + 1e-5) * w_ref[...] + b_ref[...]
    dot = lambda a, w: lax.dot_general(a, w, (((1,), (1,)), ((), ())),
                                       precision=HI)
    tq_ref[...] = (dot(ln, wq_ref[...]) + bq_ref[...]) * 0.25
    tkv_ref[:, :D] = dot(ln, wk_ref[...]) + bk_ref[...]
    tkv_ref[:, D:] = dot(ln, wv_ref[...]) + bv_ref[...]
    skip_ref[...] = dot(ln, ws_ref[...]) + bs_ref[...]


def _pre(low_emb_in, norm_w, norm_b, Wq, bq, Wk, bk, Wv, bv, Ws, bs):
    rows = 1000
    grid = N_LOW // rows
    full = pl.BlockSpec((D, D), lambda i: (0, 0))
    vec = pl.BlockSpec((1, D), lambda i: (0, 0))
    chunk = pl.BlockSpec((rows, D), lambda i: (i, 0))
    kchunk = pl.BlockSpec((rows, 2 * D), lambda i: (i, 0))
    return pl.pallas_call(
        _pre_body,
        grid=(grid,),
        in_specs=[chunk, vec, vec, full, vec, full, vec, full, vec, full, vec],
        out_specs=[chunk, kchunk, chunk],
        out_shape=[jax.ShapeDtypeStruct((N_LOW, D), jnp.float32),
                   jax.ShapeDtypeStruct((N_LOW, 2 * D), jnp.float32),
                   jax.ShapeDtypeStruct((N_LOW, D), jnp.float32)],
    )(low_emb_in, norm_w.reshape(1, D), norm_b.reshape(1, D),
      Wq, bq.reshape(1, D), Wk, bk.reshape(1, D), Wv, bv.reshape(1, D),
      Ws, bs.reshape(1, D))


def _highA_body(hs_ref, hd_ref, a_ref, acc):
    i = pl.program_id(0)

    @pl.when(i == 0)
    def _():
        acc[...] = jnp.zeros_like(acc)

    hs = hs_ref[0, 0, :]
    hd = hd_ref[0, 0, :]
    cols = lax.broadcasted_iota(jnp.int32, (1024, N_HIGH), 1)
    ms = (hs[:, None] == cols).astype(jnp.float32)
    md = (hd[:, None] == cols).astype(jnp.float32)
    acc[...] += lax.dot_general(md, ms, (((0,), (0,)), ((), ())),
                                precision=HI)

    @pl.when(i == pl.num_programs(0) - 1)
    def _():
        a_ref[...] = acc[...]


def _highA(high_edge_index):
    hs = high_edge_index[0].reshape(E_HIGH // 1024, 1, 1024)
    hd = high_edge_index[1].reshape(E_HIGH // 1024, 1, 1024)
    espec = pl.BlockSpec((1, 1, 1024), lambda i: (i, 0, 0))
    return pl.pallas_call(
        _highA_body,
        grid=(E_HIGH // 1024,),
        in_specs=[espec, espec],
        out_specs=pl.BlockSpec((N_HIGH, N_HIGH), lambda i: (0, 0)),
        out_shape=jax.ShapeDtypeStruct((N_HIGH, N_HIGH), jnp.float32),
        scratch_shapes=[pltpu.VMEM((N_HIGH, N_HIGH), jnp.float32)],
    )(hs, hd)


def _high2_body(x_ref, a_ref, w_ref, b_ref, eps_ref, ginw_ref, ginb_ref,
                wq_ref, bq_ref, wk_ref, bk_ref, wv_ref, bv_ref,
                wo_ref, bo_ref, out_ref):
    x = x_ref[...]
    mu = jnp.mean(x, axis=-1, keepdims=True)
    var = jnp.mean((x - mu) ** 2, axis=-1, keepdims=True)
    ln = (x - mu) / jnp.sqrt(var + 1e-5) * w_ref[...] + b_ref[...]
    dot = lambda a, w: lax.dot_general(a, w, (((1,), (1,)), ((), ())),
                                       precision=HI)
    agg = lax.dot_general(a_ref[...], ln, (((1,), (0,)), ((), ())),
                          precision=HI)
    gin = dot((1.0 + eps_ref[0, 0]) * ln + agg, ginw_ref[...]) + ginb_ref[...]
    q = dot(ln, wq_ref[...]) + bq_ref[...]
    k = dot(ln, wk_ref[...]) + bk_ref[...]
    v = dot(ln, wv_ref[...]) + bv_ref[...]
    outs = []
    for h in range(H):
        qh = q[:, h * DH:(h + 1) * DH]
        kh = k[:, h * DH:(h + 1) * DH]
        vh = v[:, h * DH:(h + 1) * DH]
        s = lax.dot_general(qh, kh, (((1,), (1,)), ((), ())),
                            precision=HI) * (DH ** -0.5)
        s = s - jnp.max(s, axis=-1, keepdims=True)
        e = jnp.exp(s)
        att = e / jnp.sum(e, axis=-1, keepdims=True)
        outs.append(lax.dot_general(att, vh, (((1,), (0,)), ((), ())),
                                    precision=HI))
    mh = jnp.concatenate(outs, axis=1)
    out_ref[...] = dot(mh, wo_ref[...]) + bo_ref[...] + gin


def _high2(high_emb_in, A, norm_w, norm_b, gin_eps, gin_W, gin_b,
           Wq, Wk, Wv, bq, bk, bv, Wo, bo):
    return pl.pallas_call(
        _high2_body,
        out_shape=jax.ShapeDtypeStruct((N_HIGH, D), jnp.float32),
    )(high_emb_in, A, norm_w.reshape(1, D), norm_b.reshape(1, D),
      gin_eps.reshape(1, 1), gin_W, gin_b.reshape(1, D),
      Wq, bq.reshape(1, D), Wk, bk.reshape(1, D), Wv, bv.reshape(1, D),
      Wo, bo.reshape(1, D))


def _sc_edge_body(tq_hbm, tkv_hbm, ls_hbm, ld_hbm,
                  num_hbm, den_hbm,
                  idx_s, idx_d, idx_den, qrows, kvrows, onum, dbuf,
                  ntab, dtab, sga, sgb):
    cid = lax.axis_index("c")
    sid = lax.axis_index("s")
    wid = sid * NC + cid

    lane = lax.iota(jnp.int32, 16)
    low8 = lane < 8
    zero16 = jnp.zeros((16,), jnp.float32)

    def zrow(r, carry):
        rv = jnp.full((16,), r, jnp.int32)
        for cc in range(D // 16):
            plsc.store_scatter(dbuf, [rv, cc * 16 + lane], zero16)
        return carry

    lax.fori_loop(0, C, zrow, 0)
    tbase = sid * (TROWS // NS)
    for kk in range(TROWS // NS // C):
        pltpu.sync_copy(dbuf, ntab.at[pl.ds(tbase + kk * C, C)])
    pltpu.sync_copy(dbuf.at[pl.ds(0, TROWS // NS - (TROWS // NS // C) * C)],
                    ntab.at[pl.ds(tbase + (TROWS // NS // C) * C,
                                  TROWS // NS - (TROWS // NS // C) * C)])

    @pl.when(sid == 0)
    def _():
        for kk in range(DROWS // C):
            pltpu.sync_copy(dbuf, dtab.at[pl.ds(kk * C, C)])
        pltpu.sync_copy(dbuf.at[pl.ds(0, DROWS - (DROWS // C) * C)],
                        dtab.at[pl.ds((DROWS // C) * C,
                                      DROWS - (DROWS // C) * C)])

    plsc.subcore_barrier()

    def chunk(j, carry):
        base = (j * NW + wid) * C
        pltpu.sync_copy(ls_hbm.at[pl.ds(base, C)], idx_s)
        pltpu.sync_copy(ld_hbm.at[pl.ds(base, C)], idx_d)
        c1 = pltpu.async_copy(tq_hbm.at[idx_d], qrows, sga)
        c2 = pltpu.async_copy(tkv_hbm.at[idx_s], kvrows, sgb)
        c1.wait()
        c2.wait()

        def groupA(g, gcarry):
            base16 = g * 16
            eidx = base16 + lane
            ldv = plsc.load_gather(idx_d, [eidx])
            plsc.store_scatter(idx_den, [eidx], lax.shift_right_logical(ldv, 4))
            colbv = lax.shift_left(ldv & 15, 3)
            for e in range(16):
                i = base16 + e
                sv = zero16
                for h in range(H):
                    sl = pl.ds(h * DH, DH)
                    p = qrows[i, sl] * kvrows[i, sl]
                    sv = jnp.where(lane == h, jnp.sum(p), sv)
                evm = jnp.where(low8, jnp.exp(sv), 0.0)
                plsc.store_scatter(dbuf, [jnp.full((16,), i, jnp.int32),
                                          colbv[e] + lane], evm, mask=low8)
                for h in range(H):
                    onum[i, pl.ds(h * DH, DH)] = (
                        kvrows[i, pl.ds(D + h * DH, DH)] * evm[h])
            return gcarry

        lax.fori_loop(0, C // 16, groupA, 0)

        pltpu.sync_copy(onum, ntab.at[idx_d], add=True)
        pltpu.sync_copy(dbuf, dtab.at[idx_den], add=True)

        def groupC(g, gcarry):
            eidx = g * 16 + lane
            ldv = plsc.load_gather(idx_d, [eidx])
            colb = lax.shift_left(ldv & 15, 3)
            for h in range(H):
                plsc.store_scatter(dbuf, [eidx, colb + h], zero16)
            return gcarry

        lax.fori_loop(0, C // 16, groupC, 0)
        return carry

    base_chunks = NCHUNK // NW
    extra = NCHUNK - base_chunks * NW
    nch = jnp.where(wid < extra, base_chunks + 1, base_chunks)
    lax.fori_loop(0, nch, chunk, 0)
    plsc.subcore_barrier()

    obase = sid * (TROWS // NS)
    pltpu.sync_copy(ntab.at[pl.ds(obase, TROWS // NS)],
                    num_hbm.at[cid, pl.ds(obase, TROWS // NS)])

    @pl.when(sid == 0)
    def _():
        pltpu.sync_copy(dtab, den_hbm.at[cid])


_sc_mesh = plsc.VectorSubcoreMesh(core_axis_name="c", subcore_axis_name="s")

_sc_edges = pl.kernel(
    _sc_edge_body,
    out_type=[jax.ShapeDtypeStruct((NC, TROWS, D), jnp.float32),
              jax.ShapeDtypeStruct((NC, DROWS, D), jnp.float32)],
    mesh=_sc_mesh,
    compiler_params=pltpu.CompilerParams(needs_layout_passes=False),
    scratch_types=[
        pltpu.VMEM((C,), jnp.int32),
        pltpu.VMEM((C,), jnp.int32),
        pltpu.VMEM((C,), jnp.int32),
        pltpu.VMEM((C, D), jnp.float32),
        pltpu.VMEM((C, 2 * D), jnp.float32),
        pltpu.VMEM((C, D), jnp.float32),
        pltpu.VMEM((C, D), jnp.float32),
        pltpu.VMEM_SHARED((TROWS, D), jnp.float32),
        pltpu.VMEM_SHARED((DROWS, D), jnp.float32),
        pltpu.SemaphoreType.DMA,
        pltpu.SemaphoreType.DMA,
    ],
)


def _gelu(x):
    return 0.5 * x * (1.0 + lax.erf(x * (2.0 ** -0.5)))


def _post_body(n0_ref, n1_ref, d0_ref, d1_ref, skip_ref, lb_ref, he2_ref,
               chlq_ref, chlk_ref, chlv_ref, clhq_ref, clhk_ref, clhv_ref,
               oh_ref, ol_ref, pooled, counts):
    i = pl.program_id(0)

    @pl.when(i == 0)
    def _():
        pooled[...] = jnp.zeros_like(pooled)
        counts[...] = jnp.zeros_like(counts)

    dot = lambda a, w: lax.dot_general(a, w, (((1,), (1,)), ((), ())),
                                       precision=HI)

    num = n0_ref[...] + n1_ref[...]
    den8 = d0_ref[...] + d1_ref[...]
    hrow = lax.broadcasted_iota(jnp.int32, (8, D), 0)
    hcol = lax.broadcasted_iota(jnp.int32, (8, D), 1) // DH
    expand = (hrow == hcol).astype(jnp.float32)
    den = lax.dot_general(den8, expand, (((1,), (0,)), ((), ())),
                          precision=HI)
    tout = num / jnp.maximum(den, 1e-30)
    low2 = tout + skip_ref[...]

    lb = lb_ref[0, 0, :]
    cols = lax.broadcasted_iota(jnp.int32, (lb.shape[0], N_HIGH), 1)
    P = (lb[:, None] == cols).astype(jnp.float32)
    pooled[...] += lax.dot_general(P, low2, (((0,), (0,)), ((), ())),
                                   precision=HI)
    counts[...] += lax.dot_general(P, jnp.ones_like(low2),
                                   (((0,), (0,)), ((), ())), precision=HI)

    he2 = he2_ref[...]
    hpn = lax.dot_general(P, he2, (((1,), (0,)), ((), ())), precision=HI)
    Q = dot(low2, clhq_ref[...])
    K = dot(hpn, clhk_ref[...])
    V = dot(low2, clhv_ref[...])
    w = jnp.sum(Q * K, axis=1, keepdims=True) * (float(D) ** -0.5)
    ol_ref[...] = _gelu(w * V)

    @pl.when(i == pl.num_programs(0) - 1)
    def _():
        x = pooled[...] / jnp.maximum(counts[...], 1.0)
        Qh = dot(he2, chlq_ref[...])
        Kh = dot(x, chlk_ref[...])
        Vh = dot(he2, chlv_ref[...])
        wh = jnp.sum(Qh * Kh, axis=1, keepdims=True) * (float(D) ** -0.5)
        oh_ref[...] = _gelu(wh * Vh)


def _post(num0, num1, den0, den1, skip, low_batch, high_emb2,
          chl_Q, chl_K, chl_V, clh_Q, clh_K, clh_V):
    rows = 2000
    grid = N_LOW // rows
    chunk = pl.BlockSpec((rows, D), lambda i: (i, 0))
    dchunk = pl.BlockSpec((rows, 8), lambda i: (i, 0))
    full = pl.BlockSpec((D, D), lambda i: (0, 0))
    hfull = pl.BlockSpec((N_HIGH, D), lambda i: (0, 0))
    lspec = pl.BlockSpec((1, 1, rows), lambda i: (i, 0, 0))
    lb3 = low_batch.reshape(grid, 1, rows)
    return pl.pallas_call(
        _post_body,
        grid=(grid,),
        in_specs=[chunk, chunk, dchunk, dchunk, chunk, lspec, hfull,
                  full, full, full, full, full, full],
        out_specs=[hfull, chunk],
        out_shape=[jax.ShapeDtypeStruct((N_HIGH, D), jnp.float32),
                   jax.ShapeDtypeStruct((N_LOW, D), jnp.float32)],
        scratch_shapes=[pltpu.VMEM((N_HIGH, D), jnp.float32),
                        pltpu.VMEM((N_HIGH, D), jnp.float32)],
    )(num0, num1, den0, den1, skip, lb3, high_emb2,
      chl_Q, chl_K, chl_V, clh_Q, clh_K, clh_V)


def kernel(high_emb_in, high_edge_index, low_emb_in, low_edge_index,
           low_batch, norm_w, norm_b, gin_eps, gin_W, gin_b,
           mha_Wq, mha_Wk, mha_Wv, mha_bq, mha_bk, mha_bv, mha_Wo, mha_bo,
           tc_Wq, tc_bq, tc_Wk, tc_bk, tc_Wv, tc_bv, tc_Wskip, tc_bskip,
           chl_Q, chl_K, chl_V, clh_Q, clh_K, clh_V):
    tq, tkv, skip = _pre(low_emb_in, norm_w, norm_b,
                         tc_Wq, tc_bq, tc_Wk, tc_bk, tc_Wv, tc_bv,
                         tc_Wskip, tc_bskip)
    A = _highA(high_edge_index)
    high_emb2 = _high2(high_emb_in, A, norm_w, norm_b, gin_eps, gin_W, gin_b,
                       mha_Wq, mha_Wk, mha_Wv, mha_bq, mha_bk, mha_bv,
                       mha_Wo, mha_bo)
    ls = low_edge_index[0]
    ld = low_edge_index[1]
    num_p, den_p = _sc_edges(tq, tkv, ls, ld)
    den_lin = den_p.reshape(NC, DROWS * 16, 8)
    out_high, out_low = _post(num_p[0, :N_LOW], num_p[1, :N_LOW],
                              den_lin[0, :N_LOW], den_lin[1, :N_LOW],
                              skip, low_batch, high_emb2,
                              chl_Q, chl_K, chl_V, clh_Q, clh_K, clh_V)
    return (out_high, out_low)


# speedup vs baseline: 1.4465x; 1.3363x over previous
import functools

import jax
import jax.numpy as jnp
from jax import lax
from jax.experimental import pallas as pl
from jax.experimental.pallas import tpu as pltpu
from jax.experimental.pallas import tpu_sc as plsc

D = 128
H = 8
DH = D // H
N_HIGH = 512
E_HIGH = 8192
N_LOW = 10000
E_LOW = 320000

NC = 2
NS = 16
NW = NC * NS
C = 64
CH = C // 2
NCHUNK = E_LOW // C
TROWS = 10112
DROWS = 632
HI = jax.lax.Precision.HIGHEST


def _pre_body(x_ref, w_ref, b_ref, wq_ref, bq_ref, wk_ref, bk_ref,
              wv_ref, bv_ref, ws_ref, bs_ref,
              tq_ref, tk_ref, tv_ref, skip_ref):
    x = x_ref[...]
    mu = jnp.mean(x, axis=-1, keepdims=True)
    var = jnp.mean((x - mu) ** 2, axis=-1, keepdims=True)
    ln = (x - mu) / jnp.sqrt(var + 1e-5) * w_ref[...] + b_ref[...]
    dot = lambda a, w: lax.dot_general(a, w, (((1,), (1,)), ((), ())),
                                       precision=HI)
    tq_ref[...] = (dot(ln, wq_ref[...]) + bq_ref[...]) * 0.25
    tk_ref[...] = dot(ln, wk_ref[...]) + bk_ref[...]
    tv_ref[...] = dot(ln, wv_ref[...]) + bv_ref[...]
    skip_ref[...] = dot(ln, ws_ref[...]) + bs_ref[...]


def _pre(low_emb_in, norm_w, norm_b, Wq, bq, Wk, bk, Wv, bv, Ws, bs):
    rows = 1000
    grid = N_LOW // rows
    full = pl.BlockSpec((D, D), lambda i: (0, 0))
    vec = pl.BlockSpec((1, D), lambda i: (0, 0))
    chunk = pl.BlockSpec((rows, D), lambda i: (i, 0))
    return pl.pallas_call(
        _pre_body,
        grid=(grid,),
        in_specs=[chunk, vec, vec, full, vec, full, vec, full, vec, full, vec],
        out_specs=[chunk, chunk, chunk, chunk],
        out_shape=[jax.ShapeDtypeStruct((N_LOW, D), jnp.float32)] * 4,
    )(low_emb_in, norm_w.reshape(1, D), norm_b.reshape(1, D),
      Wq, bq.reshape(1, D), Wk, bk.reshape(1, D), Wv, bv.reshape(1, D),
      Ws, bs.reshape(1, D))


def _highA_body(hs_ref, hd_ref, a_ref, acc):
    i = pl.program_id(0)

    @pl.when(i == 0)
    def _():
        acc[...] = jnp.zeros_like(acc)

    hs = hs_ref[0, 0, :]
    hd = hd_ref[0, 0, :]
    cols = lax.broadcasted_iota(jnp.int32, (1024, N_HIGH), 1)
    ms = (hs[:, None] == cols).astype(jnp.float32)
    md = (hd[:, None] == cols).astype(jnp.float32)
    acc[...] += lax.dot_general(md, ms, (((0,), (0,)), ((), ())),
                                precision=HI)

    @pl.when(i == pl.num_programs(0) - 1)
    def _():
        a_ref[...] = acc[...]


def _highA(high_edge_index):
    hs = high_edge_index[0].reshape(E_HIGH // 1024, 1, 1024)
    hd = high_edge_index[1].reshape(E_HIGH // 1024, 1, 1024)
    espec = pl.BlockSpec((1, 1, 1024), lambda i: (i, 0, 0))
    return pl.pallas_call(
        _highA_body,
        grid=(E_HIGH // 1024,),
        in_specs=[espec, espec],
        out_specs=pl.BlockSpec((N_HIGH, N_HIGH), lambda i: (0, 0)),
        out_shape=jax.ShapeDtypeStruct((N_HIGH, N_HIGH), jnp.float32),
        scratch_shapes=[pltpu.VMEM((N_HIGH, N_HIGH), jnp.float32)],
    )(hs, hd)


def _high2_body(x_ref, a_ref, w_ref, b_ref, eps_ref, ginw_ref, ginb_ref,
                wq_ref, bq_ref, wk_ref, bk_ref, wv_ref, bv_ref,
                wo_ref, bo_ref, out_ref):
    x = x_ref[...]
    mu = jnp.mean(x, axis=-1, keepdims=True)
    var = jnp.mean((x - mu) ** 2, axis=-1, keepdims=True)
    ln = (x - mu) / jnp.sqrt(var + 1e-5) * w_ref[...] + b_ref[...]
    dot = lambda a, w: lax.dot_general(a, w, (((1,), (1,)), ((), ())),
                                       precision=HI)
    agg = lax.dot_general(a_ref[...], ln, (((1,), (0,)), ((), ())),
                          precision=HI)
    gin = dot((1.0 + eps_ref[0, 0]) * ln + agg, ginw_ref[...]) + ginb_ref[...]
    q = dot(ln, wq_ref[...]) + bq_ref[...]
    k = dot(ln, wk_ref[...]) + bk_ref[...]
    v = dot(ln, wv_ref[...]) + bv_ref[...]
    outs = []
    for h in range(H):
        qh = q[:, h * DH:(h + 1) * DH]
        kh = k[:, h * DH:(h + 1) * DH]
        vh = v[:, h * DH:(h + 1) * DH]
        s = lax.dot_general(qh, kh, (((1,), (1,)), ((), ())),
                            precision=HI) * (DH ** -0.5)
        s = s - jnp.max(s, axis=-1, keepdims=True)
        e = jnp.exp(s)
        att = e / jnp.sum(e, axis=-1, keepdims=True)
        outs.append(lax.dot_general(att, vh, (((1,), (0,)), ((), ())),
                                    precision=HI))
    mh = jnp.concatenate(outs, axis=1)
    out_ref[...] = dot(mh, wo_ref[...]) + bo_ref[...] + gin


def _high2(high_emb_in, A, norm_w, norm_b, gin_eps, gin_W, gin_b,
           Wq, Wk, Wv, bq, bk, bv, Wo, bo):
    return pl.pallas_call(
        _high2_body,
        out_shape=jax.ShapeDtypeStruct((N_HIGH, D), jnp.float32),
    )(high_emb_in, A, norm_w.reshape(1, D), norm_b.reshape(1, D),
      gin_eps.reshape(1, 1), gin_W, gin_b.reshape(1, D),
      Wq, bq.reshape(1, D), Wk, bk.reshape(1, D), Wv, bv.reshape(1, D),
      Wo, bo.reshape(1, D))


def _sc_edge_body(tq_hbm, tk_hbm, tv_hbm, ls_hbm, ld_hbm,
                  num_hbm, den_hbm,
                  idx_s, idx_d, idx_den, qrows, krows, vrows, onum, dbuf,
                  ntab, dtab, sga, sgb, sgc):
    cid = lax.axis_index("c")
    sid = lax.axis_index("s")
    wid = sid * NC + cid

    lane = lax.iota(jnp.int32, 16)
    low8 = lane < 8
    zero16 = jnp.zeros((16,), jnp.float32)

    def zrow(r, carry):
        rv = jnp.full((16,), r, jnp.int32)
        for cc in range(D // 16):
            plsc.store_scatter(dbuf, [rv, cc * 16 + lane], zero16)
        return carry

    lax.fori_loop(0, C, zrow, 0)
    tbase = sid * (TROWS // NS)
    for kk in range(TROWS // NS // C):
        pltpu.sync_copy(dbuf, ntab.at[pl.ds(tbase + kk * C, C)])
    pltpu.sync_copy(dbuf.at[pl.ds(0, TROWS // NS - (TROWS // NS // C) * C)],
                    ntab.at[pl.ds(tbase + (TROWS // NS // C) * C,
                                  TROWS // NS - (TROWS // NS // C) * C)])

    @pl.when(sid == 0)
    def _():
        for kk in range(DROWS // C):
            pltpu.sync_copy(dbuf, dtab.at[pl.ds(kk * C, C)])
        pltpu.sync_copy(dbuf.at[pl.ds(0, DROWS - (DROWS // C) * C)],
                        dtab.at[pl.ds((DROWS // C) * C,
                                      DROWS - (DROWS // C) * C)])

    plsc.subcore_barrier()

    def chunk(j, carry):
        base = (j * NW + wid) * C
        pltpu.sync_copy(ls_hbm.at[pl.ds(base, C)], idx_s)
        pltpu.sync_copy(ld_hbm.at[pl.ds(base, C)], idx_d)
        c1 = pltpu.async_copy(tq_hbm.at[idx_d], qrows, sga)
        c2 = pltpu.async_copy(tk_hbm.at[idx_s], krows, sgb)
        c3 = pltpu.async_copy(tv_hbm.at[idx_s], vrows, sgc)
        c1.wait()
        c2.wait()
        c3.wait()

        def groupA(g, gcarry):
            base16 = g * 16
            eidx = base16 + lane
            ldv = plsc.load_gather(idx_d, [eidx])
            plsc.store_scatter(idx_den, [eidx], lax.shift_right_logical(ldv, 4))
            colbv = lax.shift_left(ldv & 15, 3)
            for e in range(16):
                i = base16 + e
                sv = zero16
                for h in range(H):
                    sl = pl.ds(h * DH, DH)
                    p = qrows[i, sl] * krows[i, sl]
                    sv = jnp.where(lane == h, jnp.sum(p), sv)
                evm = jnp.where(low8, jnp.exp(sv), 0.0)
                plsc.store_scatter(dbuf, [jnp.full((16,), i, jnp.int32),
                                          colbv[e] + lane], evm, mask=low8)
                for h in range(H):
                    sl = pl.ds(h * DH, DH)
                    onum[i, sl] = vrows[i, sl] * evm[h]
            return gcarry

        lax.fori_loop(0, C // 16, groupA, 0)

        pltpu.sync_copy(onum, ntab.at[idx_d], add=True)
        pltpu.sync_copy(dbuf, dtab.at[idx_den], add=True)

        def groupC(g, gcarry):
            eidx = g * 16 + lane
            ldv = plsc.load_gather(idx_d, [eidx])
            colb = lax.shift_left(ldv & 15, 3)
            for h in range(H):
                plsc.store_scatter(dbuf, [eidx, colb + h], zero16)
            return gcarry

        lax.fori_loop(0, C // 16, groupC, 0)
        return carry

    base_chunks = NCHUNK // NW
    extra = NCHUNK - base_chunks * NW
    nch = jnp.where(wid < extra, base_chunks + 1, base_chunks)
    lax.fori_loop(0, nch, chunk, 0)
    plsc.subcore_barrier()

    obase = sid * (TROWS // NS)
    pltpu.sync_copy(ntab.at[pl.ds(obase, TROWS // NS)],
                    num_hbm.at[cid, pl.ds(obase, TROWS // NS)])

    @pl.when(sid == 0)
    def _():
        pltpu.sync_copy(dtab, den_hbm.at[cid])


_sc_mesh = plsc.VectorSubcoreMesh(core_axis_name="c", subcore_axis_name="s")

_sc_edges = pl.kernel(
    _sc_edge_body,
    out_type=[jax.ShapeDtypeStruct((NC, TROWS, D), jnp.float32),
              jax.ShapeDtypeStruct((NC, DROWS, D), jnp.float32)],
    mesh=_sc_mesh,
    compiler_params=pltpu.CompilerParams(needs_layout_passes=False),
    scratch_types=[
        pltpu.VMEM((C,), jnp.int32),
        pltpu.VMEM((C,), jnp.int32),
        pltpu.VMEM((C,), jnp.int32),
        pltpu.VMEM((C, D), jnp.float32),
        pltpu.VMEM((C, D), jnp.float32),
        pltpu.VMEM((C, D), jnp.float32),
        pltpu.VMEM((C, D), jnp.float32),
        pltpu.VMEM((C, D), jnp.float32),
        pltpu.VMEM_SHARED((TROWS, D), jnp.float32),
        pltpu.VMEM_SHARED((DROWS, D), jnp.float32),
        pltpu.SemaphoreType.DMA,
        pltpu.SemaphoreType.DMA,
        pltpu.SemaphoreType.DMA,
    ],
)


def _gelu(x):
    return 0.5 * x * (1.0 + lax.erf(x * (2.0 ** -0.5)))


def _post_body(n0_ref, n1_ref, d0_ref, d1_ref, skip_ref, lb_ref, he2_ref,
               chlq_ref, chlk_ref, chlv_ref, clhq_ref, clhk_ref, clhv_ref,
               oh_ref, ol_ref, pooled, counts):
    i = pl.program_id(0)

    @pl.when(i == 0)
    def _():
        pooled[...] = jnp.zeros_like(pooled)
        counts[...] = jnp.zeros_like(counts)

    dot = lambda a, w: lax.dot_general(a, w, (((1,), (1,)), ((), ())),
                                       precision=HI)

    num = n0_ref[...] + n1_ref[...]
    den8 = d0_ref[...] + d1_ref[...]
    hrow = lax.broadcasted_iota(jnp.int32, (8, D), 0)
    hcol = lax.broadcasted_iota(jnp.int32, (8, D), 1) // DH
    expand = (hrow == hcol).astype(jnp.float32)
    den = lax.dot_general(den8, expand, (((1,), (0,)), ((), ())),
                          precision=HI)
    tout = num / jnp.maximum(den, 1e-30)
    low2 = tout + skip_ref[...]

    lb = lb_ref[0, 0, :]
    cols = lax.broadcasted_iota(jnp.int32, (lb.shape[0], N_HIGH), 1)
    P = (lb[:, None] == cols).astype(jnp.float32)
    pooled[...] += lax.dot_general(P, low2, (((0,), (0,)), ((), ())),
                                   precision=HI)
    counts[...] += lax.dot_general(P, jnp.ones_like(low2),
                                   (((0,), (0,)), ((), ())), precision=HI)

    he2 = he2_ref[...]
    hpn = lax.dot_general(P, he2, (((1,), (0,)), ((), ())), precision=HI)
    Q = dot(low2, clhq_ref[...])
    K = dot(hpn, clhk_ref[...])
    V = dot(low2, clhv_ref[...])
    w = jnp.sum(Q * K, axis=1, keepdims=True) * (float(D) ** -0.5)
    ol_ref[...] = _gelu(w * V)

    @pl.when(i == pl.num_programs(0) - 1)
    def _():
        x = pooled[...] / jnp.maximum(counts[...], 1.0)
        Qh = dot(he2, chlq_ref[...])
        Kh = dot(x, chlk_ref[...])
        Vh = dot(he2, chlv_ref[...])
        wh = jnp.sum(Qh * Kh, axis=1, keepdims=True) * (float(D) ** -0.5)
        oh_ref[...] = _gelu(wh * Vh)


def _post(num0, num1, den0, den1, skip, low_batch, high_emb2,
          chl_Q, chl_K, chl_V, clh_Q, clh_K, clh_V):
    rows = 2000
    grid = N_LOW // rows
    chunk = pl.BlockSpec((rows, D), lambda i: (i, 0))
    dchunk = pl.BlockSpec((rows, 8), lambda i: (i, 0))
    full = pl.BlockSpec((D, D), lambda i: (0, 0))
    hfull = pl.BlockSpec((N_HIGH, D), lambda i: (0, 0))
    lspec = pl.BlockSpec((1, 1, rows), lambda i: (i, 0, 0))
    lb3 = low_batch.reshape(grid, 1, rows)
    return pl.pallas_call(
        _post_body,
        grid=(grid,),
        in_specs=[chunk, chunk, dchunk, dchunk, chunk, lspec, hfull,
                  full, full, full, full, full, full],
        out_specs=[hfull, chunk],
        out_shape=[jax.ShapeDtypeStruct((N_HIGH, D), jnp.float32),
                   jax.ShapeDtypeStruct((N_LOW, D), jnp.float32)],
        scratch_shapes=[pltpu.VMEM((N_HIGH, D), jnp.float32),
                        pltpu.VMEM((N_HIGH, D), jnp.float32)],
    )(num0, num1, den0, den1, skip, lb3, high_emb2,
      chl_Q, chl_K, chl_V, clh_Q, clh_K, clh_V)


def kernel(high_emb_in, high_edge_index, low_emb_in, low_edge_index,
           low_batch, norm_w, norm_b, gin_eps, gin_W, gin_b,
           mha_Wq, mha_Wk, mha_Wv, mha_bq, mha_bk, mha_bv, mha_Wo, mha_bo,
           tc_Wq, tc_bq, tc_Wk, tc_bk, tc_Wv, tc_bv, tc_Wskip, tc_bskip,
           chl_Q, chl_K, chl_V, clh_Q, clh_K, clh_V):
    tq, tk, tv, skip = _pre(low_emb_in, norm_w, norm_b,
                            tc_Wq, tc_bq, tc_Wk, tc_bk, tc_Wv, tc_bv,
                            tc_Wskip, tc_bskip)
    A = _highA(high_edge_index)
    high_emb2 = _high2(high_emb_in, A, norm_w, norm_b, gin_eps, gin_W, gin_b,
                       mha_Wq, mha_Wk, mha_Wv, mha_bq, mha_bk, mha_bv,
                       mha_Wo, mha_bo)
    ls = low_edge_index[0]
    ld = low_edge_index[1]
    num_p, den_p = _sc_edges(tq, tk, tv, ls, ld)
    den_lin = den_p.reshape(NC, DROWS * 16, 8)
    out_high, out_low = _post(num_p[0, :N_LOW], num_p[1, :N_LOW],
                              den_lin[0, :N_LOW], den_lin[1, :N_LOW],
                              skip, low_batch, high_emb2,
                              chl_Q, chl_K, chl_V, clh_Q, clh_K, clh_V)
    return (out_high, out_low)
